# baseline (speedup 1.0000x reference)
import functools

import jax
import jax.numpy as jnp
from jax import lax
from jax.experimental import pallas as pl
from jax.experimental.pallas import tpu as pltpu
from jax.experimental.pallas import tpu_sc as plsc

B, T, H, W, C = 4, 16, 128, 128, 32
NREG = 14
NCT, NCS = 2, 25


def _sc_token_body(frames_hbm, rpos_hbm, out_hbm, rpos_v, slab_v, tok_v):
    wid = lax.axis_index("s") * 2 + lax.axis_index("c")

    @pl.when(wid == 0)
    def _():
        pltpu.sync_copy(rpos_hbm, rpos_v)
        rv = rpos_v[...]
        rb, rt, rh, rw = rv[0], rv[1], rv[2], rv[3]
        pltpu.sync_copy(frames_hbm.at[rb, rt, rh], slab_v)
        chunk = rw // 16
        pos = jnp.full((16,), rw % 16, jnp.int32)
        for c in range(C):
            acc = jnp.zeros((16,), jnp.float32)
            for j in range(W // 16):
                vj = slab_v[c, pl.ds(j * 16, 16)]
                acc = acc + jnp.where(chunk == j, vj, jnp.float32(0.0))
            tokc = acc.at[pos].get(mode="promise_in_bounds")
            tok_v[pl.ds(c * 16, 16)] = tokc
        pltpu.sync_copy(tok_v, out_hbm)


def _sc_token(frames_t, rpos16):
    fn = functools.partial(
        pl.kernel,
        out_type=jax.ShapeDtypeStruct((C * 16,), jnp.float32),
        mesh=plsc.VectorSubcoreMesh(core_axis_name="c", subcore_axis_name="s"),
        scratch_types=[
            pltpu.VMEM((16,), jnp.int32),
            pltpu.VMEM((C, W), jnp.float32),
            pltpu.VMEM((C * 16,), jnp.float32),
        ],
    )(_sc_token_body)
    return fn(frames_t, rpos16)


def _tc_body(bs_ref, ts_ref, hs_ref, ws_ref, x_ref, tok_ref, out_ref, m_ref):
    ib = pl.program_id(0)
    it = pl.program_id(1)

    hh = lax.broadcasted_iota(jnp.int32, (H, W), 0)
    ww = lax.broadcasted_iota(jnp.int32, (H, W), 1)

    any_active = jnp.int32(0)
    m1 = jnp.zeros((H, W), jnp.bool_)
    m2 = jnp.zeros((H, W), jnp.bool_)
    for i in range(NREG):
        bi = bs_ref[i]
        ti = ts_ref[i]
        t0 = jnp.maximum(ti - NCT, 0)
        t1 = jnp.minimum(ti + NCT, T - 1)
        act = (bi == ib) & (it >= t0) & (it < t1)
        any_active = any_active | act.astype(jnp.int32)
        if i == NREG - 1:
            continue

        hi = hs_ref[i]
        wi = ws_ref[i]
        h0 = jnp.maximum(hi - NCS, 0)
        h1 = jnp.minimum(hi + NCS, H - 1)
        w0 = jnp.maximum(wi - NCS, 0)
        w1 = jnp.minimum(wi + NCS, W - 1)
        box = ((hh >= h0) & (hh < h1) & (ww >= w0) & (ww < w1)) & act
        if i < NREG - 2:
            m1 = m1 | box
        else:
            m2 = box

    x = x_ref[0, 0]
    tok = tok_ref[0, 0][None]
    m1b = m1[:, None, :]
    m2b = m2[:, None, :]
    out = jnp.where(m2b, tok, jnp.where(m1b, jnp.float32(0.0), x))
    out_ref[0, 0] = out
    m_ref[0, 0, 0] = any_active


def _masked_copy(frames_t, b16, t16, h16, w16, tok):
    out, m = pl.pallas_call(
        _tc_body,
        grid=(B, T),
        in_specs=[
            pl.BlockSpec(memory_space=pltpu.SMEM),
            pl.BlockSpec(memory_space=pltpu.SMEM),
            pl.BlockSpec(memory_space=pltpu.SMEM),
            pl.BlockSpec(memory_space=pltpu.SMEM),
            pl.BlockSpec((1, 1, H, C, W), lambda i, j: (i, j, 0, 0, 0)),
            pl.BlockSpec((1, 1, C, 1), lambda i, j: (0, 0, 0, 0)),
        ],
        out_specs=[
            pl.BlockSpec((1, 1, H, C, W), lambda i, j: (i, j, 0, 0, 0)),
            pl.BlockSpec((1, 1, 1), lambda i, j: (i * T + j, 0, 0),
                         memory_space=pltpu.SMEM),
        ],
        out_shape=[
            jax.ShapeDtypeStruct((B, T, H, C, W), jnp.float32),
            jax.ShapeDtypeStruct((B * T, 1, 1), jnp.int32),
        ],
    )(b16, t16, h16, w16, frames_t, tok)
    return out, m


def kernel(frames, b, t, h, w, rpos):
    b16 = b[:16].astype(jnp.int32)
    t16 = t[:16].astype(jnp.int32)
    h16 = h[:16].astype(jnp.int32)
    w16 = w[:16].astype(jnp.int32)

    frames_t = jnp.transpose(frames, (0, 1, 2, 4, 3))

    rpos16 = jnp.pad(rpos.astype(jnp.int32), (0, 12))
    tok = _sc_token(frames_t, rpos16).reshape(C, 16)[:, 0].reshape(1, 1, C, 1)

    out_t, m = _masked_copy(frames_t, b16, t16, h16, w16, tok)
    out = jnp.transpose(out_t, (0, 1, 2, 4, 3))
    M = (m[:, 0, 0] != 0).reshape(B, T)
    return out, M


# speedup vs baseline: 1.2996x; 1.2996x over previous
import functools

import jax
import jax.numpy as jnp
from jax import lax
from jax.experimental import pallas as pl
from jax.experimental.pallas import tpu as pltpu
from jax.experimental.pallas import tpu_sc as plsc

B, T, H, W, C = 4, 16, 128, 128, 32
NREG = 14
NCT, NCS = 2, 25


def _sc_prep_body(frames_hbm, p_hbm, tok_hbm, m_hbm,
                  rpos_v, b_v, t_v, slab_v, tok_v, m_v):
    wid = lax.axis_index("s") * 2 + lax.axis_index("c")

    @pl.when(wid == 0)
    def _():
        pltpu.sync_copy(p_hbm.at[pl.ds(64, 16)], rpos_v)
        pltpu.sync_copy(p_hbm.at[pl.ds(0, 16)], b_v)
        pltpu.sync_copy(p_hbm.at[pl.ds(16, 16)], t_v)
        rv = rpos_v[...]
        rb, rt, rh, rw = rv[0], rv[1], rv[2], rv[3]
        pltpu.sync_copy(frames_hbm.at[rb, rt, rh], slab_v)
        chunk = rw // 16
        pos = jnp.full((16,), rw % 16, jnp.int32)
        for c in range(C):
            acc = jnp.zeros((16,), jnp.float32)
            for j in range(W // 16):
                vj = slab_v[c, pl.ds(j * 16, 16)]
                acc = acc + jnp.where(chunk == j, vj, jnp.float32(0.0))
            tokc = acc.at[pos].get(mode="promise_in_bounds")
            for j in range(W // 16):
                tok_v[pl.ds(c * W + j * 16, 16)] = tokc

        bv = b_v[...]
        tv = t_v[...]
        tl = lax.iota(jnp.int32, 16)
        for q in range(B):
            mq = jnp.zeros((16,), jnp.float32)
            for i in range(NREG):
                bi = bv[i]
                ti = tv[i]
                t0 = jnp.maximum(ti - NCT, 0)
                t1 = jnp.minimum(ti + NCT, T - 1)
                ht0 = jnp.where(tl >= t0, jnp.float32(1.0), jnp.float32(0.0))
                ht1 = jnp.where(tl < t1, jnp.float32(1.0), jnp.float32(0.0))
                mq = mq + jnp.where(bi == q, ht0 * ht1, jnp.float32(0.0))
            m_v[pl.ds(q * 16, 16)] = mq
        pltpu.sync_copy(tok_v, tok_hbm)
        pltpu.sync_copy(m_v, m_hbm)


def _sc_prep(frames_t, params):
    fn = functools.partial(
        pl.kernel,
        out_type=[
            jax.ShapeDtypeStruct((C * W,), jnp.float32),
            jax.ShapeDtypeStruct((B * T,), jnp.float32),
        ],
        mesh=plsc.VectorSubcoreMesh(core_axis_name="c", subcore_axis_name="s"),
        scratch_types=[
            pltpu.VMEM((16,), jnp.int32),
            pltpu.VMEM((16,), jnp.int32),
            pltpu.VMEM((16,), jnp.int32),
            pltpu.VMEM((C, W), jnp.float32),
            pltpu.VMEM((C * W,), jnp.float32),
            pltpu.VMEM((B * T,), jnp.float32),
        ],
    )(_sc_prep_body)
    return fn(frames_t, params)


TB = 4


def _tc_body(p_ref, x_ref, tok_ref, out_ref):
    ib = pl.program_id(0)
    jt = pl.program_id(1)

    hh = lax.broadcasted_iota(jnp.int32, (H, W), 0)
    ww = lax.broadcasted_iota(jnp.int32, (H, W), 1)
    tok = tok_ref[0, 0]

    for k in range(TB):
        it = jt * TB + k
        m1 = jnp.zeros((H, W), jnp.bool_)
        m2 = jnp.zeros((H, W), jnp.bool_)
        for i in range(NREG - 1):
            bi = p_ref[i]
            ti = p_ref[16 + i]
            t0 = jnp.maximum(ti - NCT, 0)
            t1 = jnp.minimum(ti + NCT, T - 1)
            act = (bi == ib) & (it >= t0) & (it < t1)
            hi = p_ref[32 + i]
            wi = p_ref[48 + i]
            h0 = jnp.maximum(hi - NCS, 0)
            h1 = jnp.minimum(hi + NCS, H - 1)
            w0 = jnp.maximum(wi - NCS, 0)
            w1 = jnp.minimum(wi + NCS, W - 1)
            box = ((hh >= h0) & (hh < h1) & (ww >= w0) & (ww < w1)) & act
            if i < NREG - 2:
                m1 = m1 | box
            else:
                m2 = box

        x = x_ref[0, k]
        m1b = m1[:, None, :]
        m2b = m2[:, None, :]
        out = jnp.where(m2b, tok, jnp.where(m1b, jnp.float32(0.0), x))
        out_ref[0, k] = out


def _masked_copy(frames_t, params, tok):
    out = pl.pallas_call(
        _tc_body,
        grid=(B, T // TB),
        in_specs=[
            pl.BlockSpec(memory_space=pltpu.SMEM),
            pl.BlockSpec((1, TB, H, C, W), lambda i, j: (i, j, 0, 0, 0)),
            pl.BlockSpec((1, 1, C, W), lambda i, j: (0, 0, 0, 0)),
        ],
        out_specs=pl.BlockSpec((1, TB, H, C, W), lambda i, j: (i, j, 0, 0, 0)),
        out_shape=jax.ShapeDtypeStruct((B, T, H, C, W), jnp.float32),
    )(params, frames_t, tok)
    return out


def kernel(frames, b, t, h, w, rpos):
    params = jnp.stack([
        b[:16].astype(jnp.int32),
        t[:16].astype(jnp.int32),
        h[:16].astype(jnp.int32),
        w[:16].astype(jnp.int32),
        jnp.pad(rpos.astype(jnp.int32), (0, 12)),
    ]).reshape(-1)

    frames_t = jnp.transpose(frames, (0, 1, 2, 4, 3))

    tok, m = _sc_prep(frames_t, params)
    tok = tok.reshape(1, 1, C, W)

    out_t = _masked_copy(frames_t, params, tok)
    out = jnp.transpose(out_t, (0, 1, 2, 4, 3))
    M = (m.reshape(B, T) != 0)
    return out, M
